# Initial kernel scaffold; baseline (speedup 1.0000x reference)
import jax
import jax.numpy as jnp
from jax.experimental import pallas as pl


def kernel(x, pos, batch, W1, b1):
    raise NotImplementedError("write your pallas kernel here")


# speedup vs baseline: 1.4233x; 1.4233x over previous
import jax
import jax.numpy as jnp
from jax.experimental import pallas as pl
from jax.experimental.pallas import tpu as pltpu

N = 320000
D = 128
S = 1024
BN = 512
NB = N // BN


def _fused_kernel(bounds_ref, x_ref, pos_ref, ids_ref, wx_ref, wp_ref, b_ref,
                  out_ref):
    i = pl.program_id(0)

    @pl.when(i == 0)
    def _init():
        out_ref[...] = jnp.zeros_like(out_ref)

    h = jnp.dot(x_ref[...], wx_ref[...], preferred_element_type=jnp.float32)
    p = pos_ref[...]
    h += p[:, 0:1] * wp_ref[0:1, :]
    h += p[:, 1:2] * wp_ref[1:2, :]
    h += p[:, 2:3] * wp_ref[2:3, :]
    h += b_ref[...]
    h = jnp.maximum(h, 0.0)

    ids = ids_ref[0]
    s_lo = bounds_ref[i, 0]
    s_hi = bounds_ref[i, 1]

    def body(s, _):
        col = jnp.max(jnp.where(ids == s, h, 0.0), axis=0, keepdims=True)
        cur = out_ref[pl.ds(s, 1), :]
        out_ref[pl.ds(s, 1), :] = jnp.maximum(cur, col)
        return 0

    jax.lax.fori_loop(s_lo, s_hi + 1, body, 0, unroll=False)


def kernel(x, pos, batch, W1, b1):
    ids = batch.astype(jnp.int32)
    bounds = jnp.stack([ids[::BN], ids[BN - 1::BN]], axis=1)
    ids3 = ids.reshape(NB, BN, 1)
    wx = W1[:D]
    wp = W1[D:]
    b = b1.reshape(1, 128)

    grid_spec = pltpu.PrefetchScalarGridSpec(
        num_scalar_prefetch=1,
        grid=(NB,),
        in_specs=[
            pl.BlockSpec((BN, D), lambda i, b_: (i, 0)),
            pl.BlockSpec((BN, 3), lambda i, b_: (i, 0)),
            pl.BlockSpec((1, BN, 1), lambda i, b_: (i, 0, 0)),
            pl.BlockSpec((D, 128), lambda i, b_: (0, 0)),
            pl.BlockSpec((3, 128), lambda i, b_: (0, 0)),
            pl.BlockSpec((1, 128), lambda i, b_: (0, 0)),
        ],
        out_specs=pl.BlockSpec((S, 128), lambda i, b_: (0, 0)),
    )

    pooled = pl.pallas_call(
        _fused_kernel,
        grid_spec=grid_spec,
        out_shape=jax.ShapeDtypeStruct((S, 128), jnp.float32),
    )(bounds, x, pos, ids3, wx, wp, b)

    pos_out = jnp.zeros((S, 3), dtype=pos.dtype)
    batch_out = jnp.arange(S, dtype=batch.dtype)
    return pooled, pos_out, batch_out
